# baseline (speedup 1.0000x reference)
import functools

import jax
import jax.numpy as jnp
from jax import lax
from jax.experimental import pallas as pl
from jax.experimental.pallas import tpu as pltpu
from jax.experimental.pallas import tpu_sc as plsc

D = 64
BATCH, SEQ = 4096, 50
B = BATCH * SEQ
SUB = 128
N_ROWS = B // SUB
NC, NS = 2, 16
NW = NC * NS
ROWS_PER_TILE = N_ROWS // NW
K = 5
N_CHUNK = ROWS_PER_TILE // K
CHUNK = K * SUB


def _build():
  mesh = plsc.VectorSubcoreMesh(core_axis_name="c", subcore_axis_name="s")

  @functools.partial(
      pl.kernel,
      mesh=mesh,
      compiler_params=pltpu.CompilerParams(use_tc_tiling_on_sc=False),
      out_type=jax.ShapeDtypeStruct((B, D), jnp.float32),
      scratch_types=[
          pltpu.VMEM((ROWS_PER_TILE, SUB), jnp.int32),
          pltpu.VMEM((2, CHUNK, D), jnp.float32),
          pltpu.SemaphoreType.DMA,
          pltpu.SemaphoreType.DMA,
      ],
  )
  def emb(idx_hbm, table_hbm, out_hbm, idx_v, rows_v, gsem, osem):
    wid = lax.axis_index("s") * NC + lax.axis_index("c")
    pltpu.sync_copy(idx_hbm.at[wid], idx_v)
    base = pl.multiple_of(wid * (ROWS_PER_TILE * SUB), CHUNK)

    def do_chunk(c, b):
      buf = rows_v.at[b]
      copies = [
          pltpu.async_copy(
              table_hbm.at[idx_v.at[c * K + k]],
              buf.at[pl.ds(k * SUB, SUB)],
              gsem,
          )
          for k in range(K)
      ]
      for cp in copies:
        cp.wait()
      off = pl.multiple_of(base + c * CHUNK, CHUNK)
      pltpu.make_async_copy(buf, out_hbm.at[pl.ds(off, CHUNK)], osem).start()

    def wait_out(b):
      pltpu.make_async_copy(
          rows_v.at[b], out_hbm.at[pl.ds(base, CHUNK)], osem).wait()

    do_chunk(0, 0)
    do_chunk(1, 1)

    def body(i, carry):
      c0 = 2 * i
      for b in range(2):
        wait_out(b)
        do_chunk(c0 + b, b)
      return carry

    lax.fori_loop(1, N_CHUNK // 2, body, 0)
    wait_out(0)
    wait_out(1)

  return emb


_emb = _build()


def kernel(item_inputs, table):
  idx = item_inputs.astype(jnp.int32).reshape(NW, ROWS_PER_TILE, SUB)
  out = _emb(idx, table)
  return out.reshape(BATCH, SEQ, D)
